# Initial kernel scaffold; baseline (speedup 1.0000x reference)
import jax
import jax.numpy as jnp
from jax.experimental import pallas as pl


def kernel(x1, edge_index1, x2, edge_index2, W0, b0, W1, b1):
    raise NotImplementedError("write your pallas kernel here")


# speedup vs baseline: 11.1073x; 11.1073x over previous
import functools

import jax
import jax.numpy as jnp
from jax import lax
from jax.experimental import pallas as pl
from jax.experimental.pallas import tpu as pltpu
from jax.experimental.pallas import tpu_sc as plsc

N_NODES = 10000
N_PAD = 10240
DIM = 128
N_EDGES = 320000
CHUNK = 128
NSC = 2
NTILES = 16
EDGES_PER_TILE = -(-N_EDGES // (NTILES * CHUNK)) * CHUNK
E_PAD = EDGES_PER_TILE * NTILES
N_CHUNKS = EDGES_PER_TILE // CHUNK
ROWS_PER_TILE = N_PAD // NTILES
SRC_FILL = 10100
DST_FILL = 10200

_mesh = plsc.VectorSubcoreMesh(core_axis_name="c", subcore_axis_name="s")


def _fill_vmem_2d(ref, rows, value):
    def row(i, _):
        for j in range(DIM // 16):
            ref[i, pl.ds(j * 16, 16)] = jnp.full((16,), value, jnp.float32)
        return 0
    lax.fori_loop(0, rows, row, 0)


@functools.partial(
    pl.kernel,
    mesh=_mesh,
    out_type=jax.ShapeDtypeStruct((NSC, N_PAD), jnp.float32),
    scratch_types=[
        pltpu.VMEM((CHUNK,), jnp.int32),
        pltpu.VMEM((CHUNK,), jnp.float32),
        pltpu.VMEM((ROWS_PER_TILE,), jnp.float32),
        pltpu.VMEM_SHARED((N_PAD,), jnp.float32),
    ],
)
def _deg_kernel(dst_hbm, deg_hbm, dst_v, ones_v, zero_v, acc_sh):
    c = lax.axis_index("c")
    s = lax.axis_index("s")

    def fill1(i, _):
        ones_v[pl.ds(i * 16, 16)] = jnp.ones((16,), jnp.float32)
        return 0
    lax.fori_loop(0, CHUNK // 16, fill1, 0)

    def fill0(i, _):
        zero_v[pl.ds(i * 16, 16)] = jnp.zeros((16,), jnp.float32)
        return 0
    lax.fori_loop(0, ROWS_PER_TILE // 16, fill0, 0)

    pltpu.sync_copy(zero_v, acc_sh.at[pl.ds(s * ROWS_PER_TILE, ROWS_PER_TILE)])
    plsc.subcore_barrier()

    tile_base = s * EDGES_PER_TILE

    def body(k, _):
        off = tile_base + k * CHUNK
        pltpu.sync_copy(dst_hbm.at[c, pl.ds(off, CHUNK)], dst_v)
        pltpu.sync_copy(ones_v, acc_sh.at[dst_v], add=True)
        return 0
    lax.fori_loop(0, N_CHUNKS, body, 0)

    plsc.subcore_barrier()
    pltpu.sync_copy(acc_sh.at[pl.ds(s * ROWS_PER_TILE, ROWS_PER_TILE)],
                    deg_hbm.at[c, pl.ds(s * ROWS_PER_TILE, ROWS_PER_TILE)])


@functools.partial(
    pl.kernel,
    mesh=_mesh,
    out_type=jax.ShapeDtypeStruct((NSC, N_PAD, DIM), jnp.float32),
    scratch_types=[
        pltpu.VMEM((CHUNK,), jnp.int32),
        pltpu.VMEM((CHUNK,), jnp.int32),
        pltpu.VMEM((CHUNK, DIM), jnp.float32),
        pltpu.VMEM_SHARED((N_PAD, DIM), jnp.float32),
        pltpu.SemaphoreType.DMA,
    ],
)
def _agg_kernel(table_hbm, src_hbm, dst_hbm, out_hbm,
                src_v, dst_v, rows_v, acc_sh, sem):
    c = lax.axis_index("c")
    s = lax.axis_index("s")

    _fill_vmem_2d(rows_v, CHUNK, 0.0)
    for j in range(ROWS_PER_TILE // CHUNK):
        pltpu.sync_copy(
            rows_v, acc_sh.at[pl.ds(s * ROWS_PER_TILE + j * CHUNK, CHUNK), :])
    plsc.subcore_barrier()

    tile_base = s * EDGES_PER_TILE

    def body(k, _):
        off = tile_base + k * CHUNK
        pltpu.sync_copy(src_hbm.at[c, pl.ds(off, CHUNK)], src_v)
        pltpu.sync_copy(dst_hbm.at[c, pl.ds(off, CHUNK)], dst_v)
        pltpu.async_copy(table_hbm.at[src_v], rows_v, sem).wait()
        pltpu.sync_copy(rows_v, acc_sh.at[dst_v], add=True)
        return 0
    lax.fori_loop(0, N_CHUNKS, body, 0)

    plsc.subcore_barrier()
    pltpu.sync_copy(acc_sh.at[pl.ds(s * ROWS_PER_TILE, ROWS_PER_TILE), :],
                    out_hbm.at[c, pl.ds(s * ROWS_PER_TILE, ROWS_PER_TILE), :])


_BLK = 256
_NBLK = NSC * N_PAD // _BLK
_BLK_PER_G = N_PAD // _BLK


def _row_spec():
    return pl.BlockSpec((_BLK, DIM), lambda i: (i, 0))


def _deg_spec():
    return pl.BlockSpec((_BLK, 1), lambda i: (i, 0))


def _full_spec():
    return pl.BlockSpec((DIM, DIM), lambda i: (0, 0))


def _tc_matmul_scale(x_ref, w_ref, deg_ref, out_ref):
    inv = lax.rsqrt(deg_ref[...] + 1.0)
    out_ref[...] = jnp.dot(x_ref[...], w_ref[...],
                           preferred_element_type=jnp.float32) * inv


def _row_mask():
    i = pl.program_id(0)
    g = i // _BLK_PER_G
    base = i * _BLK - g * N_PAD
    rows = base + lax.broadcasted_iota(jnp.int32, (_BLK, 1), 0)
    return rows < N_NODES


def _tc_layer1(s_ref, h_ref, deg_ref, w_ref, b_ref, out_ref):
    inv = lax.rsqrt(deg_ref[...] + 1.0)
    h1 = jnp.maximum((s_ref[...] + h_ref[...]) * inv + b_ref[...], 0.0)
    h1s = jnp.dot(h1, w_ref[...], preferred_element_type=jnp.float32) * inv
    out_ref[...] = jnp.where(_row_mask(), h1s, 0.0)


def _tc_stats(s_ref, h_ref, deg_ref, b_ref, sum_ref, sq_ref):
    i = pl.program_id(0)
    inv = lax.rsqrt(deg_ref[...] + 1.0)
    agg = (s_ref[...] + h_ref[...]) * inv + b_ref[...]
    agg = jnp.where(_row_mask(), agg, 0.0)

    @pl.when(i % _BLK_PER_G == 0)
    def _():
        sum_ref[...] = jnp.zeros_like(sum_ref)
        sq_ref[...] = jnp.zeros_like(sq_ref)

    sum_ref[...] += jnp.sum(agg, axis=0)[None, None, :]
    sq_ref[...] += jnp.sum(agg * agg, axis=0)[None, None, :]


def _tc_standardize(s_ref, h_ref, deg_ref, b_ref, sum_ref, sq_ref, out_ref):
    inv = lax.rsqrt(deg_ref[...] + 1.0)
    agg = (s_ref[...] + h_ref[...]) * inv + b_ref[...]
    n = jnp.float32(N_NODES)
    mean = sum_ref[0] / n
    var = (sq_ref[0] - n * mean * mean) / (n - 1.0)
    out_ref[...] = (agg - mean) * lax.rsqrt(var)


def kernel(x1, edge_index1, x2, edge_index2, W0, b0, W1, b1):
    f32 = jnp.float32
    pad_n = N_PAD - N_NODES
    x_cat = jnp.concatenate([
        x1, jnp.zeros((pad_n, DIM), f32),
        x2, jnp.zeros((pad_n, DIM), f32)], axis=0)

    pad_e = E_PAD - N_EDGES
    def prep(ei, g):
        src = jnp.concatenate(
            [ei[0], jnp.full((pad_e,), SRC_FILL, jnp.int32)]) + g * N_PAD
        dst = jnp.concatenate(
            [ei[1], jnp.full((pad_e,), DST_FILL, jnp.int32)])
        return src, dst
    s1, d1 = prep(edge_index1, 0)
    s2, d2 = prep(edge_index2, 1)
    src_cat = jnp.stack([s1, s2])
    dst_cat = jnp.stack([d1, d2])

    deg = _deg_kernel(dst_cat).reshape(NSC * N_PAD, 1)

    b0r = b0.reshape(1, DIM)
    b1r = b1.reshape(1, DIM)
    grid = (_NBLK,)

    h0s = pl.pallas_call(
        _tc_matmul_scale,
        grid=grid,
        in_specs=[_row_spec(), _full_spec(), _deg_spec()],
        out_specs=_row_spec(),
        out_shape=jax.ShapeDtypeStruct((NSC * N_PAD, DIM), f32),
    )(x_cat, W0, deg)

    s0 = _agg_kernel(h0s, src_cat, dst_cat).reshape(NSC * N_PAD, DIM)

    bias_spec = pl.BlockSpec((1, DIM), lambda i: (0, 0))
    h1s = pl.pallas_call(
        _tc_layer1,
        grid=grid,
        in_specs=[_row_spec(), _row_spec(), _deg_spec(), _full_spec(), bias_spec],
        out_specs=_row_spec(),
        out_shape=jax.ShapeDtypeStruct((NSC * N_PAD, DIM), f32),
    )(s0, h0s, deg, W1, b0r)

    s1agg = _agg_kernel(h1s, src_cat, dst_cat).reshape(NSC * N_PAD, DIM)

    stat_spec = pl.BlockSpec((1, 1, DIM), lambda i: (i // _BLK_PER_G, 0, 0))
    colsum, colsq = pl.pallas_call(
        _tc_stats,
        grid=grid,
        in_specs=[_row_spec(), _row_spec(), _deg_spec(), bias_spec],
        out_specs=[stat_spec, stat_spec],
        out_shape=[jax.ShapeDtypeStruct((NSC, 1, DIM), f32)] * 2,
    )(s1agg, h1s, deg, b1r)

    z = pl.pallas_call(
        _tc_standardize,
        grid=grid,
        in_specs=[_row_spec(), _row_spec(), _deg_spec(), bias_spec,
                  stat_spec, stat_spec],
        out_specs=_row_spec(),
        out_shape=jax.ShapeDtypeStruct((NSC * N_PAD, DIM), f32),
    )(s1agg, h1s, deg, b1r, colsum, colsq)

    return z[:N_NODES], z[N_PAD:N_PAD + N_NODES]
